# baseline (speedup 1.0000x reference)
import functools

import jax
import jax.numpy as jnp
from jax import lax
from jax.experimental import pallas as pl
from jax.experimental.pallas import tpu as pltpu
from jax.experimental.pallas import tpu_sc as plsc

NUM_CORES = 2
NUM_SUBCORES = 16
NUM_WORKERS = NUM_CORES * NUM_SUBCORES
LANES = 16
BATCH = 16384
D = 32
ROWS_PER_TILE = 8
NTILES = 1000000 // ROWS_PER_TILE
BPW = BATCH // NUM_WORKERS
CH = 128
NPASS = BPW // CH


def _mf_body(uidx_hbm, iidx_hbm, utab_hbm, itab_hbm, utab2_hbm, itab2_hbm,
             out_hbm, uidx_v, iidx_v, ubuf_v, ibuf_v, out_v, sem):
    wid = lax.axis_index("s") * NUM_CORES + lax.axis_index("c")
    base = wid * BPW

    pltpu.sync_copy(uidx_hbm.at[pl.ds(base, BPW)], uidx_v)
    pltpu.sync_copy(iidx_hbm.at[pl.ds(base, BPW)], iidx_v)

    row_iota = lax.iota(jnp.int32, LANES)

    def pass_body(p, _):
        def fire(b, _):
            cu = uidx_v[pl.ds(p * CH + b * LANES, LANES)]
            ci = iidx_v[pl.ds(p * CH + b * LANES, LANES)]
            for k in range(LANES):
                slot = b * LANES + k
                ru = cu[k]
                ri = ci[k]
                if k % 8 < 5:
                    pltpu.async_copy(
                        utab_hbm.at[lax.shift_right_logical(ru, 3), ru & 7],
                        ubuf_v.at[slot], sem)
                    pltpu.async_copy(
                        itab_hbm.at[lax.shift_right_logical(ri, 3), ri & 7],
                        ibuf_v.at[slot], sem)
                else:
                    pltpu.async_copy(utab2_hbm.at[ru], ubuf_v.at[slot], sem)
                    pltpu.async_copy(itab2_hbm.at[ri], ibuf_v.at[slot], sem)
            return 0

        lax.fori_loop(0, CH // LANES, fire, 0)
        pltpu.make_async_copy(utab_hbm.at[pl.ds(0, CH), 0], ubuf_v,
                              sem).wait()
        pltpu.make_async_copy(itab_hbm.at[pl.ds(0, CH), 0], ibuf_v,
                              sem).wait()

        def block_body(b, _):
            islot = b * LANES + row_iota

            def col_body(j, acc):
                jv = jnp.zeros((LANES,), jnp.int32) + j
                u = plsc.load_gather(ubuf_v, [islot, jv])
                v = plsc.load_gather(ibuf_v, [islot, jv])
                return acc + u * v

            acc = lax.fori_loop(0, D, col_body,
                                jnp.zeros((LANES,), jnp.float32))
            out_v[pl.ds(p * CH + b * LANES, LANES)] = (
                1.0 / (1.0 + jnp.exp(-acc)))
            return 0

        lax.fori_loop(0, CH // LANES, block_body, 0)
        return 0

    lax.fori_loop(0, NPASS, pass_body, 0)

    pltpu.sync_copy(out_v, out_hbm.at[pl.ds(base, BPW)])


_mf_kernel = functools.partial(
    pl.kernel,
    out_type=jax.ShapeDtypeStruct((BATCH,), jnp.float32),
    mesh=plsc.VectorSubcoreMesh(core_axis_name="c", subcore_axis_name="s"),
    scratch_types=[
        pltpu.VMEM((BPW,), jnp.int32),
        pltpu.VMEM((BPW,), jnp.int32),
        pltpu.VMEM((CH, D), jnp.float32),
        pltpu.VMEM((CH, D), jnp.float32),
        pltpu.VMEM((BPW,), jnp.float32),
        pltpu.SemaphoreType.DMA,
    ],
    compiler_params=pltpu.CompilerParams(needs_layout_passes=False,
                                         use_tc_tiling_on_sc=True),
)(_mf_body)


@jax.jit
def kernel(user_input, item_input, user_table, item_table):
    return _mf_kernel(
        user_input.astype(jnp.int32),
        item_input.astype(jnp.int32),
        user_table.reshape(NTILES, ROWS_PER_TILE, D),
        item_table.reshape(NTILES, ROWS_PER_TILE, D),
        *jax.lax.optimization_barrier((user_table, item_table)))


# speedup vs baseline: 2.0184x; 2.0184x over previous
import functools

import jax
import jax.numpy as jnp
from jax import lax
from jax.experimental import pallas as pl
from jax.experimental.pallas import tpu as pltpu
from jax.experimental.pallas import tpu_sc as plsc

NUM_CORES = 2
NUM_SUBCORES = 16
NUM_WORKERS = NUM_CORES * NUM_SUBCORES
LANES = 16
BATCH = 16384
D = 32
ROWS_PER_TILE = 8
NTILES = 1000000 // ROWS_PER_TILE
BPW = BATCH // NUM_WORKERS
CH = 128
NPASS = BPW // CH


def _mf_body(uidx_hbm, iidx_hbm, utab_hbm, itab_hbm, out_hbm,
             uidx_v, iidx_v, ubuf_v, ibuf_v, out_v, sem):
    wid = lax.axis_index("s") * NUM_CORES + lax.axis_index("c")
    base = wid * BPW

    pltpu.sync_copy(uidx_hbm.at[pl.ds(base, BPW)], uidx_v)
    pltpu.sync_copy(iidx_hbm.at[pl.ds(base, BPW)], iidx_v)

    row_iota = lax.iota(jnp.int32, LANES)

    def pass_body(p, _):
        def fire(b, _):
            cu = uidx_v[pl.ds(p * CH + b * LANES, LANES)]
            ci = iidx_v[pl.ds(p * CH + b * LANES, LANES)]
            for k in range(LANES):
                slot = b * LANES + k
                ru = cu[k]
                pltpu.async_copy(
                    utab_hbm.at[lax.shift_right_logical(ru, 3), ru & 7],
                    ubuf_v.at[slot], sem)
                ri = ci[k]
                pltpu.async_copy(
                    itab_hbm.at[lax.shift_right_logical(ri, 3), ri & 7],
                    ibuf_v.at[slot], sem)
            return 0

        lax.fori_loop(0, CH // LANES, fire, 0)
        pltpu.make_async_copy(utab_hbm.at[pl.ds(0, CH), 0], ubuf_v,
                              sem).wait()
        pltpu.make_async_copy(itab_hbm.at[pl.ds(0, CH), 0], ibuf_v,
                              sem).wait()

        def block_body(b, _):
            islot = b * LANES + row_iota

            def col_body(j, acc):
                jv = jnp.zeros((LANES,), jnp.int32) + j
                u = plsc.load_gather(ubuf_v, [islot, jv])
                v = plsc.load_gather(ibuf_v, [islot, jv])
                return acc + u * v

            acc = lax.fori_loop(0, D, col_body,
                                jnp.zeros((LANES,), jnp.float32))
            out_v[pl.ds(p * CH + b * LANES, LANES)] = (
                1.0 / (1.0 + jnp.exp(-acc)))
            return 0

        lax.fori_loop(0, CH // LANES, block_body, 0)
        return 0

    lax.fori_loop(0, NPASS, pass_body, 0)

    pltpu.sync_copy(out_v, out_hbm.at[pl.ds(base, BPW)])


_mf_kernel = functools.partial(
    pl.kernel,
    out_type=jax.ShapeDtypeStruct((BATCH,), jnp.float32),
    mesh=plsc.VectorSubcoreMesh(core_axis_name="c", subcore_axis_name="s"),
    scratch_types=[
        pltpu.VMEM((BPW,), jnp.int32),
        pltpu.VMEM((BPW,), jnp.int32),
        pltpu.VMEM((CH, D), jnp.float32),
        pltpu.VMEM((CH, D), jnp.float32),
        pltpu.VMEM((BPW,), jnp.float32),
        pltpu.SemaphoreType.DMA,
    ],
    compiler_params=pltpu.CompilerParams(needs_layout_passes=False,
                                         use_tc_tiling_on_sc=True),
)(_mf_body)


@jax.jit
def kernel(user_input, item_input, user_table, item_table):
    return _mf_kernel(
        user_input.astype(jnp.int32),
        item_input.astype(jnp.int32),
        user_table.reshape(NTILES, ROWS_PER_TILE, D),
        item_table.reshape(NTILES, ROWS_PER_TILE, D))
